# baseline (speedup 1.0000x reference)
import functools

import jax
import jax.numpy as jnp
from jax import lax
from jax.experimental import pallas as pl
from jax.experimental.pallas import tpu as pltpu
from jax.experimental.pallas import tpu_sc as plsc

_NC = 2
_NS = 16
_NW = _NC * _NS
_L = 16
_BB = 128
_Q = 256


@functools.lru_cache(maxsize=None)
def _make_gather(b, t, vocab):
    nq = 4
    tpad = vocab % 64
    nct = b // _BB
    thalf = t // 2
    nsu = thalf * 2
    mesh = plsc.VectorSubcoreMesh(core_axis_name="c", subcore_axis_name="s")

    @functools.partial(
        pl.kernel,
        mesh=mesh,
        compiler_params=pltpu.CompilerParams(needs_layout_passes=False),
        out_type=jax.ShapeDtypeStruct((t, vocab, b), jnp.float32),
        scratch_types=[
            pltpu.VMEM((32, _BB), jnp.int32),
            [pltpu.VMEM((_BB, 2, 128), jnp.float32) for _ in range(2)],
            [pltpu.VMEM((64, _BB + 1), jnp.float32) for _ in range(2)],
            [pltpu.VMEM((_BB,), jnp.int32) for _ in range(2)],
            [pltpu.SemaphoreType.DMA for _ in range(2)],
            [pltpu.SemaphoreType.DMA for _ in range(2)],
        ],
    )
    def gather(idxc_hbm, tq_hbm, out_hbm, idx_v, inb, outb, idxr, gsem, wsem):
        wid = lax.axis_index("s") * _NC + lax.axis_index("c")
        h = wid & 1
        ct = (wid >> 1) & 7
        tg = wid >> 4
        boff = pl.multiple_of(ct * _BB, 8)
        pltpu.sync_copy(idxc_hbm.at[ct, pl.ds(pl.multiple_of(24 * tg, 8), 32)],
                        idx_v)

        rv = [lax.iota(jnp.int32, _L) + _L * k for k in range(8)]

        def build_idx(su, p):
            tloc = tg + lax.div(su, 2)
            qbase = (2 * h + p) * vocab
            for k in range(8):
                idxr[p][pl.ds(_L * k, _L)] = idx_v[tloc, pl.ds(_L * k, _L)] + qbase

        def issue_gather(p):
            pltpu.async_copy(tq_hbm.at[idxr[p]], inb[p], gsem[p])

        def wait_gather(p):
            pltpu.make_async_copy(tq_hbm.at[idxr[p]], inb[p], gsem[p]).wait()

        def wr_descs(su, p, s, rows):
            tt = tg * thalf + lax.div(su, 2)
            voff = pl.multiple_of(512 * h + _Q * p + 64 * s, 8)
            return (outb[s % 2].at[pl.ds(0, rows), pl.ds(0, _BB)],
                    out_hbm.at[tt, pl.ds(voff, rows), pl.ds(boff, _BB)])

        def issue_write(su, p, s, rows):
            src, dst = wr_descs(su, p, s, rows)
            pltpu.async_copy(src, dst, wsem[s % 2])

        def wait_write(su, p, s, rows):
            src, dst = wr_descs(su, p, s, rows)
            pltpu.make_async_copy(src, dst, wsem[s % 2]).wait()

        def transpose_panel(p, s):
            half, sub = s // 2, s % 2

            @plsc.parallel_loop(0, _BB, step=2)
            def _row(r0):
                for r in (r0, r0 + 1):
                    rs = jnp.full((_L,), r, jnp.int32)
                    for k in range(4):
                        v = inb[p][r, half, pl.ds(64 * sub + _L * k, _L)]
                        plsc.store_scatter(outb[sub], [rv[k], rs], v)

        def wr_branch(fn, su, p, s):
            if not (p == 1 and s == 3):
                fn(su, p, s, 64)
            else:
                @pl.when(h == 0)
                def _():
                    fn(su, p, s, 64)

                @pl.when(h == 1)
                def _():
                    fn(su, p, s, tpad)

        build_idx(0, 0)
        issue_gather(0)
        build_idx(1, 1)
        issue_gather(1)

        @pl.loop(0, nsu, step=2)
        def _body(j0):
            for p in range(2):
                su = j0 + p
                wait_gather(p)
                for s in range(4):
                    if s < 2:
                        @pl.when(su >= 1)
                        def _():
                            wr_branch(wait_write, su - 1, 1 - p, s + 2)
                    else:
                        wr_branch(wait_write, su, p, s - 2)

                    transpose_panel(p, s)
                    wr_branch(issue_write, su, p, s)

                @pl.when(su + 2 < nsu)
                def _():
                    build_idx(su + 2, p)
                    issue_gather(p)

        for s in (2, 3):
            wr_branch(wait_write, nsu - 1, 1, s)

    return gather


def kernel(idx, table):
    b, t = idx.shape
    vocab = table.shape[1]
    vp = (vocab + 127) // 128 * 128
    idxc = (jnp.pad(idx.astype(jnp.int32).T, ((0, -t % 8), (0, 0)))
            .reshape(-1, b // _BB, _BB).transpose(1, 0, 2))
    tq = (jnp.pad(table, ((0, 0), (0, vp - vocab)))
          .reshape(vocab, 4, 2, 128).transpose(1, 0, 2, 3)
          .reshape(4 * vocab, 2, 128))
    out3 = _make_gather(b, t, vocab)(idxc, tq)
    return jnp.transpose(out3, (2, 0, 1))


# speedup vs baseline: 1.4809x; 1.4809x over previous
import functools

import jax
import jax.numpy as jnp
from jax import lax
from jax.experimental import pallas as pl
from jax.experimental.pallas import tpu as pltpu
from jax.experimental.pallas import tpu_sc as plsc

_NC = 2
_NS = 16
_L = 16
_BB = 128
_Q = 256


@functools.lru_cache(maxsize=None)
def _make_gather(b, t, vocab):
    thalf = t // 2
    nsu = thalf * 2
    nct = b // _BB
    mesh = plsc.VectorSubcoreMesh(core_axis_name="c", subcore_axis_name="s")

    @functools.partial(
        pl.kernel,
        mesh=mesh,
        compiler_params=pltpu.CompilerParams(needs_layout_passes=False),
        out_type=jax.ShapeDtypeStruct((t, nct, 4, 2, _BB, 128), jnp.float32),
        scratch_types=[
            pltpu.VMEM((32, _BB), jnp.int32),
            [pltpu.VMEM((_BB, 2, 128), jnp.float32) for _ in range(2)],
            [pltpu.VMEM((_BB,), jnp.int32) for _ in range(2)],
            [pltpu.SemaphoreType.DMA for _ in range(2)],
            [pltpu.SemaphoreType.DMA for _ in range(2)],
        ],
    )
    def gather(idxc_hbm, tq_hbm, mid_hbm, idx_v, inb, idxr, gsem, wsem):
        wid = lax.axis_index("s") * _NC + lax.axis_index("c")
        h = wid & 1
        ct = (wid >> 1) & 7
        tg = wid >> 4
        pltpu.sync_copy(idxc_hbm.at[ct, pl.ds(pl.multiple_of(24 * tg, 8), 32)],
                        idx_v)

        def build_idx(su, p):
            tloc = tg + lax.div(su, 2)
            qbase = (2 * h + p) * vocab
            for k in range(8):
                idxr[p][pl.ds(_L * k, _L)] = idx_v[tloc, pl.ds(_L * k, _L)] + qbase

        def issue_gather(p):
            pltpu.async_copy(tq_hbm.at[idxr[p]], inb[p], gsem[p])

        def wait_gather(p):
            pltpu.make_async_copy(tq_hbm.at[idxr[p]], inb[p], gsem[p]).wait()

        def wr_descs(su, p, s):
            tt = tg * thalf + lax.div(su, 2)
            return (inb[p].at[:, s, :], mid_hbm.at[tt, ct, 2 * h + p, s])

        def issue_writes(su, p):
            for s in range(2):
                src, dst = wr_descs(su, p, s)
                pltpu.async_copy(src, dst, wsem[p])

        def wait_writes(su, p):
            for s in range(2):
                src, dst = wr_descs(su, p, s)
                pltpu.make_async_copy(src, dst, wsem[p]).wait()

        build_idx(0, 0)
        issue_gather(0)
        build_idx(1, 1)
        issue_gather(1)

        @pl.loop(0, nsu, step=2)
        def _body(j0):
            for p in range(2):
                su = j0 + p
                wait_gather(p)
                issue_writes(su, p)

                @pl.when(su + 2 < nsu)
                def _():
                    wait_writes(su, p)
                    build_idx(su + 2, p)
                    issue_gather(p)

        for p in range(2):
            wait_writes(nsu - 2 + p, p)

    return gather


def _tc_transpose_body(mid_ref, o_ref):
    x = mid_ref[0, 0]
    xt = jnp.transpose(x, (0, 1, 3, 2))
    o_ref[0] = xt.reshape(-1, _BB)[:o_ref.shape[1]]


@functools.lru_cache(maxsize=None)
def _make_tc_transpose(b, t, vocab):
    nct = b // _BB
    return pl.pallas_call(
        _tc_transpose_body,
        grid=(t, nct),
        in_specs=[pl.BlockSpec((1, 1, 4, 2, _BB, 128),
                               lambda tt, ct: (tt, ct, 0, 0, 0, 0))],
        out_specs=pl.BlockSpec((1, vocab, _BB), lambda tt, ct: (tt, 0, ct)),
        out_shape=jax.ShapeDtypeStruct((t, vocab, b), jnp.float32),
    )


def kernel(idx, table):
    b, t = idx.shape
    vocab = table.shape[1]
    vp = (vocab + 127) // 128 * 128
    idxc = (jnp.pad(idx.astype(jnp.int32).T, ((0, -t % 8), (0, 0)))
            .reshape(-1, b // _BB, _BB).transpose(1, 0, 2))
    tq = (jnp.pad(table, ((0, 0), (0, vp - vocab)))
          .reshape(vocab, 4, 2, 128).transpose(1, 0, 2, 3)
          .reshape(4 * vocab, 2, 128))
    mid = _make_gather(b, t, vocab)(idxc, tq)
    out3 = _make_tc_transpose(b, t, vocab)(mid)
    return jnp.transpose(out3, (2, 0, 1))


# speedup vs baseline: 1.5631x; 1.0555x over previous
import functools

import jax
import jax.numpy as jnp
from jax import lax
from jax.experimental import pallas as pl
from jax.experimental.pallas import tpu as pltpu
from jax.experimental.pallas import tpu_sc as plsc

_NC = 2
_NS = 16
_L = 16
_BB = 128
_Q = 256


@functools.lru_cache(maxsize=None)
def _make_gather(b, t, vocab, toff, tsub):
    thalf = tsub // 2
    nsu = thalf * 2
    nct = b // _BB
    mesh = plsc.VectorSubcoreMesh(core_axis_name="c", subcore_axis_name="s")

    @functools.partial(
        pl.kernel,
        mesh=mesh,
        compiler_params=pltpu.CompilerParams(needs_layout_passes=False),
        out_type=jax.ShapeDtypeStruct((tsub, nct, 4, 2, _BB, 128), jnp.float32),
        scratch_types=[
            pltpu.VMEM((32, _BB), jnp.int32),
            [pltpu.VMEM((_BB, 2, 128), jnp.float32) for _ in range(2)],
            [pltpu.VMEM((_BB,), jnp.int32) for _ in range(2)],
            [pltpu.SemaphoreType.DMA for _ in range(2)],
            [pltpu.SemaphoreType.DMA for _ in range(2)],
        ],
    )
    def gather(idxc_hbm, tq_hbm, mid_hbm, idx_v, inb, idxr, gsem, wsem):
        wid = lax.axis_index("s") * _NC + lax.axis_index("c")
        h = wid & 1
        ct = (wid >> 1) & 7
        tg = wid >> 4
        t0 = toff + thalf * tg
        base = lax.div(t0, 8) * 8
        pltpu.sync_copy(idxc_hbm.at[ct, pl.ds(pl.multiple_of(base, 8), 32)],
                        idx_v)

        def build_idx(su, p):
            tloc = t0 - base + lax.div(su, 2)
            qbase = (2 * h + p) * vocab
            for k in range(8):
                idxr[p][pl.ds(_L * k, _L)] = idx_v[tloc, pl.ds(_L * k, _L)] + qbase

        def issue_gather(p):
            pltpu.async_copy(tq_hbm.at[idxr[p]], inb[p], gsem[p])

        def wait_gather(p):
            pltpu.make_async_copy(tq_hbm.at[idxr[p]], inb[p], gsem[p]).wait()

        def wr_descs(su, p, s):
            tt = tg * thalf + lax.div(su, 2)
            return (inb[p].at[:, s, :], mid_hbm.at[tt, ct, 2 * h + p, s])

        def issue_writes(su, p):
            for s in range(2):
                src, dst = wr_descs(su, p, s)
                pltpu.async_copy(src, dst, wsem[p])

        def wait_writes(su, p):
            for s in range(2):
                src, dst = wr_descs(su, p, s)
                pltpu.make_async_copy(src, dst, wsem[p]).wait()

        build_idx(0, 0)
        issue_gather(0)
        build_idx(1, 1)
        issue_gather(1)

        @pl.loop(0, nsu, step=2)
        def _body(j0):
            for p in range(2):
                su = j0 + p
                wait_gather(p)
                issue_writes(su, p)

                @pl.when(su + 2 < nsu)
                def _():
                    wait_writes(su, p)
                    build_idx(su + 2, p)
                    issue_gather(p)

        for p in range(2):
            wait_writes(nsu - 2 + p, p)

    return gather


def _tc_transpose_body(mid_ref, o_ref):
    x = mid_ref[0, 0]
    xt = jnp.transpose(x, (0, 1, 3, 2))
    o_ref[0] = xt.reshape(-1, _BB)[:o_ref.shape[1]]


def _tc_transpose_alias_body(mid_ref, prev_ref, o_ref):
    del prev_ref
    _tc_transpose_body(mid_ref, o_ref)


@functools.lru_cache(maxsize=None)
def _make_tc_transpose(b, t, vocab, toff, tsub, aliased):
    nct = b // _BB
    mid_spec = pl.BlockSpec((1, 1, 4, 2, _BB, 128),
                            lambda tt, ct: (tt, ct, 0, 0, 0, 0))
    out_spec = pl.BlockSpec((1, vocab, _BB),
                            lambda tt, ct: (toff + tt, 0, ct))
    if aliased:
        return pl.pallas_call(
            _tc_transpose_alias_body,
            grid=(tsub, nct),
            in_specs=[mid_spec, pl.BlockSpec(memory_space=pl.ANY)],
            out_specs=out_spec,
            out_shape=jax.ShapeDtypeStruct((t, vocab, b), jnp.float32),
            input_output_aliases={1: 0},
        )
    return pl.pallas_call(
        _tc_transpose_body,
        grid=(tsub, nct),
        in_specs=[mid_spec],
        out_specs=out_spec,
        out_shape=jax.ShapeDtypeStruct((t, vocab, b), jnp.float32),
    )


def kernel(idx, table):
    b, t = idx.shape
    vocab = table.shape[1]
    vp = (vocab + 127) // 128 * 128
    idxc = (jnp.pad(idx.astype(jnp.int32).T, ((0, -t % 8), (0, 0)))
            .reshape(-1, b // _BB, _BB).transpose(1, 0, 2))
    tq = (jnp.pad(table, ((0, 0), (0, vp - vocab)))
          .reshape(vocab, 4, 2, 128).transpose(1, 0, 2, 3)
          .reshape(4 * vocab, 2, 128))
    ta = t // 2 // 2 * 2
    tb = t - ta
    mid_a = _make_gather(b, t, vocab, 0, ta)(idxc, tq)
    mid_b = _make_gather(b, t, vocab, ta, tb)(idxc, tq)
    out3 = _make_tc_transpose(b, t, vocab, 0, ta, False)(mid_a)
    out3 = _make_tc_transpose(b, t, vocab, ta, tb, True)(mid_b, out3)
    return jnp.transpose(out3, (2, 0, 1))


# speedup vs baseline: 1.5980x; 1.0223x over previous
import functools

import jax
import jax.numpy as jnp
from jax import lax
from jax.experimental import pallas as pl
from jax.experimental.pallas import tpu as pltpu
from jax.experimental.pallas import tpu_sc as plsc

_NC = 2
_NS = 16
_L = 16
_BB = 128
_Q = 256


@functools.lru_cache(maxsize=None)
def _make_gather(b, t, vocab, toff, tsub):
    thalf = tsub // 2
    nsu = thalf * 2
    nct = b // _BB
    mesh = plsc.VectorSubcoreMesh(core_axis_name="c", subcore_axis_name="s")

    @functools.partial(
        pl.kernel,
        mesh=mesh,
        compiler_params=pltpu.CompilerParams(needs_layout_passes=False),
        out_type=jax.ShapeDtypeStruct((tsub, nct, 4, 2, _BB, 128), jnp.float32),
        scratch_types=[
            pltpu.VMEM((32, _BB), jnp.int32),
            [pltpu.VMEM((_BB, 2, 128), jnp.float32) for _ in range(2)],
            [pltpu.VMEM((_BB,), jnp.int32) for _ in range(2)],
            [pltpu.SemaphoreType.DMA for _ in range(2)],
            [pltpu.SemaphoreType.DMA for _ in range(2)],
        ],
    )
    def gather(idxc_hbm, tq_hbm, mid_hbm, idx_v, inb, idxr, gsem, wsem):
        wid = lax.axis_index("s") * _NC + lax.axis_index("c")
        h = wid & 1
        ct = (wid >> 1) & 7
        tg = wid >> 4
        t0 = toff + thalf * tg
        base = lax.div(t0, 8) * 8
        pltpu.sync_copy(idxc_hbm.at[ct, pl.ds(pl.multiple_of(base, 8), 32)],
                        idx_v)

        def build_idx(su, p):
            tloc = t0 - base + lax.div(su, 2)
            qbase = (2 * h + p) * vocab
            for k in range(8):
                idxr[p][pl.ds(_L * k, _L)] = idx_v[tloc, pl.ds(_L * k, _L)] + qbase

        def issue_gather(p):
            pltpu.async_copy(tq_hbm.at[idxr[p]], inb[p], gsem[p])

        def wait_gather(p):
            pltpu.make_async_copy(tq_hbm.at[idxr[p]], inb[p], gsem[p]).wait()

        def wr_descs(su, p, s):
            tt = tg * thalf + lax.div(su, 2)
            return (inb[p].at[:, s, :], mid_hbm.at[tt, ct, 2 * h + p, s])

        def issue_writes(su, p):
            for s in range(2):
                src, dst = wr_descs(su, p, s)
                pltpu.async_copy(src, dst, wsem[p])

        def wait_writes(su, p):
            for s in range(2):
                src, dst = wr_descs(su, p, s)
                pltpu.make_async_copy(src, dst, wsem[p]).wait()

        build_idx(0, 0)
        issue_gather(0)
        build_idx(1, 1)
        issue_gather(1)

        @pl.loop(0, nsu, step=2)
        def _body(j0):
            for p in range(2):
                su = j0 + p
                wait_gather(p)
                issue_writes(su, p)

                @pl.when(su + 2 < nsu)
                def _():
                    wait_writes(su, p)
                    build_idx(su + 2, p)
                    issue_gather(p)

        for p in range(2):
            wait_writes(nsu - 2 + p, p)

    return gather


def _tc_transpose_body(mid_ref, o_ref):
    x = mid_ref[0, 0]
    xt = jnp.transpose(x, (0, 1, 3, 2))
    o_ref[0] = xt.reshape(-1, _BB)[:o_ref.shape[1]]


def _tc_transpose_alias_body(mid_ref, prev_ref, o_ref):
    del prev_ref
    _tc_transpose_body(mid_ref, o_ref)


@functools.lru_cache(maxsize=None)
def _make_tc_transpose(b, t, vocab, toff, tsub, aliased):
    nct = b // _BB
    mid_spec = pl.BlockSpec((1, 1, 4, 2, _BB, 128),
                            lambda tt, ct: (tt, ct, 0, 0, 0, 0))
    out_spec = pl.BlockSpec((1, vocab, _BB),
                            lambda tt, ct: (toff + tt, 0, ct))
    if aliased:
        return pl.pallas_call(
            _tc_transpose_alias_body,
            grid=(tsub, nct),
            in_specs=[mid_spec, pl.BlockSpec(memory_space=pl.ANY)],
            out_specs=out_spec,
            out_shape=jax.ShapeDtypeStruct((t, vocab, b), jnp.float32),
            input_output_aliases={1: 0},
        )
    return pl.pallas_call(
        _tc_transpose_body,
        grid=(tsub, nct),
        in_specs=[mid_spec],
        out_specs=out_spec,
        out_shape=jax.ShapeDtypeStruct((t, vocab, b), jnp.float32),
    )


def kernel(idx, table):
    b, t = idx.shape
    vocab = table.shape[1]
    vp = (vocab + 127) // 128 * 128
    idxc = (jnp.pad(idx.astype(jnp.int32).T, ((0, -t % 8), (0, 0)))
            .reshape(-1, b // _BB, _BB).transpose(1, 0, 2))
    tq = (jnp.pad(table, ((0, 0), (0, vp - vocab)))
          .reshape(vocab, 4, 2, 128).transpose(1, 0, 2, 3)
          .reshape(4 * vocab, 2, 128))
    nchain = 4
    step = t // nchain // 2 * 2
    sizes = [step] * (nchain - 1) + [t - step * (nchain - 1)]
    mids, offs, toff = [], [], 0
    for tsub in sizes:
        mids.append(_make_gather(b, t, vocab, toff, tsub)(idxc, tq))
        offs.append(toff)
        toff += tsub
    out3 = _make_tc_transpose(b, t, vocab, offs[0], sizes[0], False)(mids[0])
    for i in range(1, nchain):
        out3 = _make_tc_transpose(b, t, vocab, offs[i], sizes[i], True)(
            mids[i], out3)
    return jnp.transpose(out3, (2, 0, 1))
